# baseline (speedup 1.0000x reference)
import functools

import jax
import jax.numpy as jnp
from jax.experimental import pallas as pl
from jax.experimental.pallas import tpu as pltpu

_K = 512
_W = 128


def _vq_kernel(x_ref, cba_ref, codes_ref, idx_ref, *, tile):
    D = x_ref.shape[1]
    xb = x_ref[0].reshape(D, tile)
    cb = cba_ref[...]
    scores = jax.lax.dot_general(
        cb, xb, (((1,), (0,)), ((), ())),
        preferred_element_type=jnp.float32)
    half_sqr = 0.5 * jnp.sum(cb * cb, axis=1)
    scores = scores - half_sqr[:, None]
    maxval = jnp.max(scores, axis=0, keepdims=True)
    kvec = jax.lax.broadcasted_iota(jnp.int32, (_K, 1), 0).astype(jnp.float32)
    gm = jnp.concatenate([cb, kvec], axis=1)
    half = _K // 4
    out = sum(
        jax.lax.dot_general(
            gm[c * half:(c + 1) * half],
            (scores[c * half:(c + 1) * half] == maxval).astype(jnp.float32),
            (((0,), (0,)), ((), ())),
            preferred_element_type=jnp.float32)
        for c in range(4))
    codes_ref[0] = out[:-1].reshape(D, tile // _W, _W)
    idx_ref[0] = out[-1].astype(jnp.int32).reshape(tile // _W, _W)


def kernel(x, codebook):
    B, D, H, W = x.shape
    N = H * W
    tile = 4096
    rows = tile // W
    grid = (B, H // rows)
    codes, idx = pl.pallas_call(
        functools.partial(_vq_kernel, tile=tile),
        grid=grid,
        compiler_params=pltpu.CompilerParams(
            dimension_semantics=("parallel", "parallel")),
        in_specs=[
            pl.BlockSpec((1, D, rows, W), lambda b, t: (b, 0, t, 0)),
            pl.BlockSpec((_K, D), lambda b, t: (0, 0)),
        ],
        out_specs=[
            pl.BlockSpec((1, D, rows, W), lambda b, t: (b, 0, t, 0)),
            pl.BlockSpec((1, rows, W), lambda b, t: (b, t, 0)),
        ],
        out_shape=[
            jax.ShapeDtypeStruct((B, D, H, W), jnp.float32),
            jax.ShapeDtypeStruct((B, H, W), jnp.int32),
        ],
    )(x, codebook)
    return codes, idx


# speedup vs baseline: 1.0746x; 1.0746x over previous
import functools

import jax
import jax.numpy as jnp
from jax.experimental import pallas as pl
from jax.experimental.pallas import tpu as pltpu

_K = 512
_W = 128


def _vq_kernel(x_ref, cba_ref, codes_ref, idx_ref, *, tile):
    D = x_ref.shape[1]
    xb = x_ref[0].reshape(D, tile)
    cb = cba_ref[...]
    scores = jax.lax.dot_general(
        cb, xb, (((1,), (0,)), ((), ())),
        preferred_element_type=jnp.float32)
    half_sqr = 0.5 * jnp.sum(cb * cb, axis=1)
    scores = scores - half_sqr[:, None]
    maxval = jnp.max(scores, axis=0, keepdims=True)
    kvec = jax.lax.broadcasted_iota(jnp.int32, (_K, 1), 0).astype(jnp.float32)
    gm = jnp.concatenate([cb, kvec], axis=1)
    half = _K // 4
    out = sum(
        jax.lax.dot_general(
            gm[c * half:(c + 1) * half],
            (scores[c * half:(c + 1) * half] == maxval).astype(jnp.float32),
            (((0,), (0,)), ((), ())),
            preferred_element_type=jnp.float32)
        for c in range(4))
    codes_ref[0] = out[:-1].reshape(D, tile // _W, _W)
    idx_ref[0] = out[-1].astype(jnp.int32).reshape(tile // _W, _W)


def kernel(x, codebook):
    B, D, H, W = x.shape
    N = H * W
    tile = 8192
    rows = tile // W
    grid = (B, H // rows)
    codes, idx = pl.pallas_call(
        functools.partial(_vq_kernel, tile=tile),
        grid=grid,
        compiler_params=pltpu.CompilerParams(
            dimension_semantics=("parallel", "parallel")),
        in_specs=[
            pl.BlockSpec((1, D, rows, W), lambda b, t: (b, 0, t, 0)),
            pl.BlockSpec((_K, D), lambda b, t: (0, 0)),
        ],
        out_specs=[
            pl.BlockSpec((1, D, rows, W), lambda b, t: (b, 0, t, 0)),
            pl.BlockSpec((1, rows, W), lambda b, t: (b, t, 0)),
        ],
        out_shape=[
            jax.ShapeDtypeStruct((B, D, H, W), jnp.float32),
            jax.ShapeDtypeStruct((B, H, W), jnp.int32),
        ],
    )(x, codebook)
    return codes, idx
